# baseline (speedup 1.0000x reference)
import functools

import jax
import jax.numpy as jnp
from jax import lax
from jax.experimental import pallas as pl
from jax.experimental.pallas import tpu as pltpu
from jax.experimental.pallas import tpu_sc as plsc

_K = 100
_NC, _NS, _L = 2, 16, 16
_NW = _NC * _NS
_NEG_INF = float("-inf")
_INT_MIN = -2147483648

_C = 160
_VPC = _C // _L
_SEG = 4000
_RING = 4
_CNT_LIM = 115
_NSPLIT = 8


def _key_of(x):
    b = plsc.bitcast(x, jnp.int32)
    return b ^ (lax.shift_right_arithmetic(b, 31) & jnp.int32(0x7FFFFFFF))


def _val_of(k):
    b = k ^ (lax.shift_right_arithmetic(k, 31) & jnp.int32(0x7FFFFFFF))
    return plsc.bitcast(b, jnp.float32)


def _sc_body(v, rows_per_w, x_hbm, tgt_hbm, s_out, d_out,
             row_buf, cmax_f, ckey_b, cand_b, tgt_v, s_acc, d_acc,
             *sems):
    nchunk = v // _C
    nseg = v // _SEG
    chunks_per_seg = _SEG // _C
    ncvec = (nchunk + _L - 1) // _L

    wid = lax.axis_index("c") * _NS + lax.axis_index("s")
    row0 = wid * rows_per_w

    iota = lax.iota(jnp.int32, _L)
    one_v = iota * 0 + 1
    ninf_v = jnp.full((_L,), _NEG_INF, jnp.float32)
    zero_f = jnp.zeros((_L,), jnp.float32)
    zero_i = iota * 0
    perms = tuple(iota ^ k for k in (1, 2, 4, 8))

    pltpu.sync_copy(tgt_hbm.at[pl.ds(row0, rows_per_w)], tgt_v)

    def _issue(base, s, k):
        pltpu.async_copy(x_hbm.at[pl.ds(base + s * _SEG, _SEG)],
                         row_buf.at[pl.ds(s * _SEG, _SEG)], sems[k])

    def _wait(base, s, k):
        pltpu.make_async_copy(x_hbm.at[pl.ds(base + s * _SEG, _SEG)],
                              row_buf.at[pl.ds(s * _SEG, _SEG)],
                              sems[k]).wait()

    def row_body(i, accs):
        sacc, dacc = accs
        base = (row0 + i) * v
        tvec = tgt_v[pl.ds((i // _L) * _L, _L)]
        tgt = tvec[zero_i + i % _L][0]

        for s in range(nseg):
            _issue(base, jnp.int32(s), s)

        def chunk_fn(s):
            def chunk_body(ci, acc):
                c = s * chunks_per_seg + ci
                off = c * _C
                macc = row_buf[pl.ds(off, _L)]
                for j in range(1, _VPC):
                    macc = jnp.maximum(macc, row_buf[pl.ds(off + j * _L, _L)])
                for p in perms:
                    macc = jnp.maximum(macc, macc[p])
                lane = c % _L
                acc2 = jnp.where(iota == lane, macc, acc)
                @pl.when(lane == _L - 1)
                def _():
                    cmax_f[pl.ds((c // _L) * _L, _L)] = acc2
                return jnp.where(lane == _L - 1, ninf_v, acc2)
            return chunk_body

        acc_last = ninf_v
        for s in range(nseg):
            _wait(base, jnp.int32(s), s)
            acc_last = lax.fori_loop(0, chunks_per_seg, chunk_fn(s), acc_last)
        cmax_f[pl.ds((nchunk // _L) * _L, _L)] = acc_last

        truth_vec = row_buf[pl.ds((tgt // _L) * _L, _L)]
        truth = truth_vec[zero_i + tgt % _L][0]
        c_t = tgt // _C
        off_t = c_t * _C
        macc = ninf_v
        for j in range(_VPC):
            xo = off_t + j * _L
            xv = row_buf[pl.ds(xo, _L)]
            macc = jnp.maximum(macc, jnp.where(xo + iota == tgt, _NEG_INF, xv))
        for p in perms:
            macc = jnp.maximum(macc, macc[p])
        g_t = (c_t // _L) * _L
        cvec = cmax_f[pl.ds(g_t, _L)]
        cmax_f[pl.ds(g_t, _L)] = jnp.where(iota == c_t % _L, macc, cvec)

        def keyify(g, mv):
            cv = cmax_f[pl.ds(g * _L, _L)]
            ckey_b[pl.ds(g * _L, _L)] = _key_of(cv)
            return jnp.maximum(mv, cv)
        mvec = lax.fori_loop(0, ncvec, keyify, ninf_v)
        for p in perms:
            mvec = jnp.maximum(mvec, mvec[p])
        m = mvec[0]

        def t_cond(st):
            it, _, cnt_v = st
            return jnp.logical_and(it < 32, cnt_v[0] > _CNT_LIM)

        def t_step(st):
            it, t_v, cnt_v = st
            trial_v = t_v + jnp.left_shift(one_v, 31 - it)
            def cnt_body(g, cv):
                kv = ckey_b[pl.ds(g * _L, _L)]
                return cv + plsc.all_reduce_population_count(kv >= trial_v)
            c2 = lax.fori_loop(0, ncvec, cnt_body, zero_i)
            ok = c2 >= _K
            return (it + 1, jnp.where(ok, trial_v, t_v), jnp.where(ok, c2, cnt_v))

        _, t_v, _ = lax.while_loop(t_cond, t_step,
                                   (jnp.int32(0), zero_i + jnp.int32(_INT_MIN),
                                    zero_i + jnp.int32(2 * nchunk)))

        def rescan_group(g, nc):
            kv = ckey_b[pl.ds(g * _L, _L)]
            qual0 = kv > t_v

            def any_left(st):
                return plsc.all_reduce_population_count(st[0])[0] > 0

            def do_lane(st):
                qual, nc = st
                lane_v = plsc.all_reduce_ffs(qual)
                off = (g * _L + lane_v[0]) * _C
                for j in range(_VPC):
                    xo = off + j * _L
                    xv = row_buf[pl.ds(xo, _L)]
                    kx = _key_of(xv)
                    msk = jnp.logical_and(kx > t_v, xo + iota != tgt)
                    plsc.store_compressed(cand_b.at[pl.ds(nc, _L)], kx, mask=msk)
                    nc = nc + plsc.all_reduce_population_count(msk)[0]
                return jnp.logical_and(qual, iota != lane_v), nc

            _, nc = lax.while_loop(any_left, do_lane, (qual0, nc))
            return nc

        ncand = lax.fori_loop(0, ncvec, rescan_group, jnp.int32(0))

        t_splat = zero_i + t_v
        for j in range(8):
            cand_b[pl.ds(ncand + j * _L, _L)] = t_splat
        nvec = (ncand + 8 * _L) // _L

        def tau_step(it, tau_v):
            trial_v = tau_v + jnp.left_shift(one_v, 31 - it)
            def cnt_body(q, cv):
                kv = cand_b[pl.ds(q * _L, _L)]
                return cv + plsc.all_reduce_population_count(kv >= trial_v)
            c2 = lax.fori_loop(0, nvec, cnt_body, zero_i)
            return jnp.where(c2 >= _K, trial_v, tau_v)
        tau_v = lax.fori_loop(0, 32, tau_step, zero_i + jnp.int32(_INT_MIN))

        def sum_body(q, st):
            sv, cv = st
            kv = cand_b[pl.ds(q * _L, _L)]
            gt = kv > tau_v
            xv = _val_of(kv)
            ev = jnp.exp(jnp.where(gt, xv - m, _NEG_INF))
            return sv + ev, cv + plsc.all_reduce_population_count(gt)
        sv, cv = lax.fori_loop(0, nvec, sum_body, (zero_f, zero_i))
        for p in perms:
            sv = sv + sv[p]
        s1 = sv[0]
        n_gt = cv[0]

        tau_val = _val_of(tau_v)[0]
        extras_in = jnp.where(iota == 0, tau_val - m,
                              jnp.where(iota == 1, truth - m, _NEG_INF))
        wts = jnp.where(iota == 0, (_K - n_gt).astype(jnp.float32),
                        jnp.where(iota == 1, 1.0, 0.0))
        ev = jnp.exp(extras_in) * wts
        for p in perms:
            ev = ev + ev[p]
        s_row = s1 + ev[0]
        d_row = m - truth

        lane_i = i % _L
        sacc2 = jnp.where(iota == lane_i, s_row, sacc)
        dacc2 = jnp.where(iota == lane_i, d_row, dacc)
        @pl.when(lane_i == _L - 1)
        def _():
            s_acc[pl.ds((i // _L) * _L, _L)] = sacc2
            d_acc[pl.ds((i // _L) * _L, _L)] = dacc2
        keep = lane_i != _L - 1
        return jnp.where(keep, sacc2, zero_f), jnp.where(keep, dacc2, zero_f)

    lax.fori_loop(0, rows_per_w, row_body, (zero_f, zero_f))

    pltpu.sync_copy(s_acc, s_out.at[pl.ds(row0, rows_per_w)])
    pltpu.sync_copy(d_acc, d_out.at[pl.ds(row0, rows_per_w)])


def _reduce_tc(s_ref, d_ref, out_ref):
    loss = jnp.log(s_ref[...]) + d_ref[...]
    tot = jnp.sum(jnp.sum(loss, axis=1, keepdims=True), axis=0, keepdims=True)
    out_ref[...] = tot / loss.size


def kernel(input, target):
    b, v = input.shape
    assert v % _SEG == 0 and _SEG % _C == 0 and b % (_NW * _NSPLIT * _L) == 0
    rows_per_w = b // (_NW * _NSPLIT)

    mesh = plsc.VectorSubcoreMesh(core_axis_name="c", subcore_axis_name="s")

    sc = pl.kernel(
        functools.partial(_sc_body, v, rows_per_w),
        out_type=[jax.ShapeDtypeStruct((b // _NSPLIT,), jnp.float32),
                  jax.ShapeDtypeStruct((b // _NSPLIT,), jnp.float32)],
        mesh=mesh,
        compiler_params=pltpu.CompilerParams(needs_layout_passes=False),
        scratch_types=[
            pltpu.VMEM((v,), jnp.float32),
            pltpu.VMEM((640,), jnp.float32),
            pltpu.VMEM((640,), jnp.int32),
            pltpu.VMEM((18560,), jnp.int32),
            pltpu.VMEM((rows_per_w,), jnp.int32),
            pltpu.VMEM((rows_per_w,), jnp.float32),
            pltpu.VMEM((rows_per_w,), jnp.float32),
        ] + [pltpu.SemaphoreType.DMA] * (v // _SEG),
    )
    s_parts, d_parts = [], []
    for g in range(_NSPLIT):
        rows = b // _NSPLIT
        xg = input[g * rows:(g + 1) * rows].reshape(rows * v)
        tg = target[g * rows:(g + 1) * rows]
        s_g, d_g = sc(xg, tg)
        s_parts.append(s_g)
        d_parts.append(d_g)
    s_arr = jnp.concatenate(s_parts)
    d_arr = jnp.concatenate(d_parts)

    out = pl.pallas_call(
        _reduce_tc,
        grid=(1,),
        in_specs=[pl.BlockSpec((_NW, b // _NW), lambda i: (0, 0)),
                  pl.BlockSpec((_NW, b // _NW), lambda i: (0, 0))],
        out_specs=pl.BlockSpec((1, 1), lambda i: (0, 0)),
        out_shape=jax.ShapeDtypeStruct((1, 1), jnp.float32),
    )(s_arr.reshape(_NW, b // _NW), d_arr.reshape(_NW, b // _NW))
    return out[0, 0]


# speedup vs baseline: 1.0337x; 1.0337x over previous
import functools

import jax
import jax.numpy as jnp
from jax import lax
from jax.experimental import pallas as pl
from jax.experimental.pallas import tpu as pltpu
from jax.experimental.pallas import tpu_sc as plsc

_K = 100
_NC, _NS, _L = 2, 16, 16
_NW = _NC * _NS
_NEG_INF = float("-inf")
_INT_MIN = -2147483648

_C = 160
_VPC = _C // _L
_SEG = 4000
_RING = 4
_CNT_LIM = 115
_NSPLIT = 8


def _key_of(x):
    b = plsc.bitcast(x, jnp.int32)
    return b ^ (lax.shift_right_arithmetic(b, 31) & jnp.int32(0x7FFFFFFF))


def _val_of(k):
    b = k ^ (lax.shift_right_arithmetic(k, 31) & jnp.int32(0x7FFFFFFF))
    return plsc.bitcast(b, jnp.float32)


def _sc_body(v, rows_per_w, x_hbm, tgt_hbm, s_out, d_out,
             row_buf, cmax_f, ckey_b, cand_b, tgt_v, s_acc, d_acc,
             *sems):
    nchunk = v // _C
    nseg = v // _SEG
    chunks_per_seg = _SEG // _C
    ncvec = (nchunk + _L - 1) // _L

    wid = lax.axis_index("c") * _NS + lax.axis_index("s")
    row0 = wid * rows_per_w

    iota = lax.iota(jnp.int32, _L)
    one_v = iota * 0 + 1
    ninf_v = jnp.full((_L,), _NEG_INF, jnp.float32)
    zero_f = jnp.zeros((_L,), jnp.float32)
    zero_i = iota * 0
    perms = tuple(iota ^ k for k in (1, 2, 4, 8))

    pltpu.sync_copy(tgt_hbm.at[pl.ds(row0, rows_per_w)], tgt_v)

    def _issue(base, s, k):
        pltpu.async_copy(x_hbm.at[pl.ds(base + s * _SEG, _SEG)],
                         row_buf.at[pl.ds(s * _SEG, _SEG)], sems[k])

    def _wait(base, s, k):
        pltpu.make_async_copy(x_hbm.at[pl.ds(base + s * _SEG, _SEG)],
                              row_buf.at[pl.ds(s * _SEG, _SEG)],
                              sems[k]).wait()

    def row_body(i, accs):
        sacc, dacc = accs
        base = (row0 + i) * v
        tvec = tgt_v[pl.ds((i // _L) * _L, _L)]
        tgt = tvec[zero_i + i % _L][0]

        def chunk_fn(s):
            def chunk_body(ci, acc):
                c = s * chunks_per_seg + ci
                off = c * _C
                macc = row_buf[pl.ds(off, _L)]
                for j in range(1, _VPC):
                    macc = jnp.maximum(macc, row_buf[pl.ds(off + j * _L, _L)])
                for p in perms:
                    macc = jnp.maximum(macc, macc[p])
                lane = c % _L
                acc2 = jnp.where(iota == lane, macc, acc)
                @pl.when(lane == _L - 1)
                def _():
                    cmax_f[pl.ds((c // _L) * _L, _L)] = acc2
                return jnp.where(lane == _L - 1, ninf_v, acc2)
            return chunk_body

        acc_last = ninf_v
        for s in range(nseg):
            _wait(base, jnp.int32(s), s)
            acc_last = lax.fori_loop(0, chunks_per_seg, chunk_fn(s), acc_last)
        cmax_f[pl.ds((nchunk // _L) * _L, _L)] = acc_last

        truth_vec = row_buf[pl.ds((tgt // _L) * _L, _L)]
        truth = truth_vec[zero_i + tgt % _L][0]
        c_t = tgt // _C
        off_t = c_t * _C
        macc = ninf_v
        for j in range(_VPC):
            xo = off_t + j * _L
            xv = row_buf[pl.ds(xo, _L)]
            macc = jnp.maximum(macc, jnp.where(xo + iota == tgt, _NEG_INF, xv))
        for p in perms:
            macc = jnp.maximum(macc, macc[p])
        g_t = (c_t // _L) * _L
        cvec = cmax_f[pl.ds(g_t, _L)]
        cmax_f[pl.ds(g_t, _L)] = jnp.where(iota == c_t % _L, macc, cvec)

        def keyify(g, mv):
            cv = cmax_f[pl.ds(g * _L, _L)]
            ckey_b[pl.ds(g * _L, _L)] = _key_of(cv)
            return jnp.maximum(mv, cv)
        mvec = lax.fori_loop(0, ncvec, keyify, ninf_v)
        for p in perms:
            mvec = jnp.maximum(mvec, mvec[p])
        m = mvec[0]

        def t_cond(st):
            it, _, cnt_v = st
            return jnp.logical_and(it < 32, cnt_v[0] > _CNT_LIM)

        def t_step(st):
            it, t_v, cnt_v = st
            trial_v = t_v + jnp.left_shift(one_v, 31 - it)
            def cnt_body(g, cv):
                kv = ckey_b[pl.ds(g * _L, _L)]
                return cv + plsc.all_reduce_population_count(kv >= trial_v)
            c2 = lax.fori_loop(0, ncvec, cnt_body, zero_i)
            ok = c2 >= _K
            return (it + 1, jnp.where(ok, trial_v, t_v), jnp.where(ok, c2, cnt_v))

        _, t_v, _ = lax.while_loop(t_cond, t_step,
                                   (jnp.int32(0), zero_i + jnp.int32(_INT_MIN),
                                    zero_i + jnp.int32(2 * nchunk)))

        def rescan_group(g, nc):
            kv = ckey_b[pl.ds(g * _L, _L)]
            qual0 = kv > t_v

            def any_left(st):
                return plsc.all_reduce_population_count(st[0])[0] > 0

            def do_lane(st):
                qual, nc = st
                lane_v = plsc.all_reduce_ffs(qual)
                off = (g * _L + lane_v[0]) * _C
                for j in range(_VPC):
                    xo = off + j * _L
                    xv = row_buf[pl.ds(xo, _L)]
                    kx = _key_of(xv)
                    msk = jnp.logical_and(kx > t_v, xo + iota != tgt)
                    plsc.store_compressed(cand_b.at[pl.ds(nc, _L)], kx, mask=msk)
                    nc = nc + plsc.all_reduce_population_count(msk)[0]
                return jnp.logical_and(qual, iota != lane_v), nc

            _, nc = lax.while_loop(any_left, do_lane, (qual0, nc))
            return nc

        ncand = lax.fori_loop(0, ncvec, rescan_group, jnp.int32(0))

        @pl.when(i + 1 < rows_per_w)
        def _():
            for s in range(nseg):
                _issue(base + v, jnp.int32(s), s)

        t_splat = zero_i + t_v
        for j in range(8):
            cand_b[pl.ds(ncand + j * _L, _L)] = t_splat
        nvec = (ncand + 8 * _L) // _L

        def tau_step(it, tau_v):
            trial_v = tau_v + jnp.left_shift(one_v, 31 - it)
            def cnt_body(q, cv):
                kv = cand_b[pl.ds(q * _L, _L)]
                return cv + plsc.all_reduce_population_count(kv >= trial_v)
            c2 = lax.fori_loop(0, nvec, cnt_body, zero_i)
            return jnp.where(c2 >= _K, trial_v, tau_v)
        tau_v = lax.fori_loop(0, 32, tau_step, zero_i + jnp.int32(_INT_MIN))

        def sum_body(q, st):
            sv, cv = st
            kv = cand_b[pl.ds(q * _L, _L)]
            gt = kv > tau_v
            xv = _val_of(kv)
            ev = jnp.exp(jnp.where(gt, xv - m, _NEG_INF))
            return sv + ev, cv + plsc.all_reduce_population_count(gt)
        sv, cv = lax.fori_loop(0, nvec, sum_body, (zero_f, zero_i))
        for p in perms:
            sv = sv + sv[p]
        s1 = sv[0]
        n_gt = cv[0]

        tau_val = _val_of(tau_v)[0]
        extras_in = jnp.where(iota == 0, tau_val - m,
                              jnp.where(iota == 1, truth - m, _NEG_INF))
        wts = jnp.where(iota == 0, (_K - n_gt).astype(jnp.float32),
                        jnp.where(iota == 1, 1.0, 0.0))
        ev = jnp.exp(extras_in) * wts
        for p in perms:
            ev = ev + ev[p]
        s_row = s1 + ev[0]
        d_row = m - truth

        lane_i = i % _L
        sacc2 = jnp.where(iota == lane_i, s_row, sacc)
        dacc2 = jnp.where(iota == lane_i, d_row, dacc)
        @pl.when(lane_i == _L - 1)
        def _():
            s_acc[pl.ds((i // _L) * _L, _L)] = sacc2
            d_acc[pl.ds((i // _L) * _L, _L)] = dacc2
        keep = lane_i != _L - 1
        return jnp.where(keep, sacc2, zero_f), jnp.where(keep, dacc2, zero_f)

    for s in range(nseg):
        _issue(row0 * v, jnp.int32(s), s)
    lax.fori_loop(0, rows_per_w, row_body, (zero_f, zero_f))

    pltpu.sync_copy(s_acc, s_out.at[pl.ds(row0, rows_per_w)])
    pltpu.sync_copy(d_acc, d_out.at[pl.ds(row0, rows_per_w)])


def _reduce_tc(s_ref, d_ref, out_ref):
    loss = jnp.log(s_ref[...]) + d_ref[...]
    tot = jnp.sum(jnp.sum(loss, axis=1, keepdims=True), axis=0, keepdims=True)
    out_ref[...] = tot / loss.size


def kernel(input, target):
    b, v = input.shape
    assert v % _SEG == 0 and _SEG % _C == 0 and b % (_NW * _NSPLIT * _L) == 0
    rows_per_w = b // (_NW * _NSPLIT)

    mesh = plsc.VectorSubcoreMesh(core_axis_name="c", subcore_axis_name="s")

    sc = pl.kernel(
        functools.partial(_sc_body, v, rows_per_w),
        out_type=[jax.ShapeDtypeStruct((b // _NSPLIT,), jnp.float32),
                  jax.ShapeDtypeStruct((b // _NSPLIT,), jnp.float32)],
        mesh=mesh,
        compiler_params=pltpu.CompilerParams(needs_layout_passes=False),
        scratch_types=[
            pltpu.VMEM((v,), jnp.float32),
            pltpu.VMEM((640,), jnp.float32),
            pltpu.VMEM((640,), jnp.int32),
            pltpu.VMEM((18560,), jnp.int32),
            pltpu.VMEM((rows_per_w,), jnp.int32),
            pltpu.VMEM((rows_per_w,), jnp.float32),
            pltpu.VMEM((rows_per_w,), jnp.float32),
        ] + [pltpu.SemaphoreType.DMA] * (v // _SEG),
    )
    s_parts, d_parts = [], []
    for g in range(_NSPLIT):
        rows = b // _NSPLIT
        xg = input[g * rows:(g + 1) * rows].reshape(rows * v)
        tg = target[g * rows:(g + 1) * rows]
        s_g, d_g = sc(xg, tg)
        s_parts.append(s_g)
        d_parts.append(d_g)
    s_arr = jnp.concatenate(s_parts)
    d_arr = jnp.concatenate(d_parts)

    out = pl.pallas_call(
        _reduce_tc,
        grid=(1,),
        in_specs=[pl.BlockSpec((_NW, b // _NW), lambda i: (0, 0)),
                  pl.BlockSpec((_NW, b // _NW), lambda i: (0, 0))],
        out_specs=pl.BlockSpec((1, 1), lambda i: (0, 0)),
        out_shape=jax.ShapeDtypeStruct((1, 1), jnp.float32),
    )(s_arr.reshape(_NW, b // _NW), d_arr.reshape(_NW, b // _NW))
    return out[0, 0]
